# Initial kernel scaffold; baseline (speedup 1.0000x reference)
import jax
import jax.numpy as jnp
from jax.experimental import pallas as pl


def kernel(h, x, edges, vel, edge_attr, emb_w, emb_b, edge_w1, edge_b1, edge_w2, edge_b2, node_w1, node_b1, node_w2, node_b2, coord_w1, coord_b1, coord_w2, vel_w1, vel_b1, vel_w2, vel_b2):
    raise NotImplementedError("write your pallas kernel here")


# speedup vs baseline: 2.5556x; 2.5556x over previous
import functools

import jax
import jax.numpy as jnp
from jax import lax
from jax.experimental import pallas as pl
from jax.experimental.pallas import tpu as pltpu
from jax.experimental.pallas import tpu_sc as plsc

_N = 10000
_E = 320000
_H = 128
_CP = 16
_L = 4

_NC = 2
_NS = 16
_NW = _NC * _NS
_EC = 80
_EPW = _E // _NW
_NCH = _EPW // _EC

_BN = 1000
_BE = 1000

_f32 = jnp.float32


def _silu(v):
    return v * jax.nn.sigmoid(v)


def _emb_body(h_ref, w_ref, b_ref, o_ref):
    o_ref[...] = jnp.dot(h_ref[...], w_ref[...],
                         preferred_element_type=_f32) + b_ref[...]


def _emb_call(h, w, b):
    g = h.shape[0] // _BN
    return pl.pallas_call(
        _emb_body,
        grid=(g,),
        in_specs=[
            pl.BlockSpec((_BN, _H), lambda i: (i, 0)),
            pl.BlockSpec((_H, _H), lambda i: (0, 0)),
            pl.BlockSpec((1, _H), lambda i: (0, 0)),
        ],
        out_specs=pl.BlockSpec((_BN, _H), lambda i: (i, 0)),
        out_shape=jax.ShapeDtypeStruct((h.shape[0], _H), _f32),
    )(h, w, b)


def _pre_body(h_ref, wr_ref, wc_ref, b1_ref, a_ref, b_ref):
    h = h_ref[...]
    a_ref[...] = jnp.dot(h, wr_ref[...], preferred_element_type=_f32) + b1_ref[...]
    b_ref[...] = jnp.dot(h, wc_ref[...], preferred_element_type=_f32)


def _pre_call(h1, w_row, w_col, b1):
    g = h1.shape[0] // _BN
    return pl.pallas_call(
        _pre_body,
        grid=(g,),
        in_specs=[
            pl.BlockSpec((_BN, _H), lambda i: (i, 0)),
            pl.BlockSpec((_H, _H), lambda i: (0, 0)),
            pl.BlockSpec((_H, _H), lambda i: (0, 0)),
            pl.BlockSpec((1, _H), lambda i: (0, 0)),
        ],
        out_specs=[
            pl.BlockSpec((_BN, _H), lambda i: (i, 0)),
            pl.BlockSpec((_BN, _H), lambda i: (i, 0)),
        ],
        out_shape=[
            jax.ShapeDtypeStruct((h1.shape[0], _H), _f32),
            jax.ShapeDtypeStruct((h1.shape[0], _H), _f32),
        ],
    )(h1, w_row, w_col, b1)


def _edge_body(pa_ref, pb_ref, cd_ref, attr_ref,
               wr_ref, wattr_ref, w2_ref, b2_ref, cw1_ref, cb1_ref, cw2_ref,
               ef_ref, tr_ref):
    cd = cd_ref[...]
    radial = jnp.sum(cd * cd, axis=1, keepdims=True)
    pre = (pa_ref[...] + pb_ref[...]
           + radial * wr_ref[...]
           + jnp.dot(attr_ref[...], wattr_ref[...], preferred_element_type=_f32))
    t1 = _silu(pre)
    ef = _silu(jnp.dot(t1, w2_ref[...], preferred_element_type=_f32) + b2_ref[...])
    t3 = _silu(jnp.dot(ef, cw1_ref[...], preferred_element_type=_f32) + cb1_ref[...])
    cm = jnp.sum(t3 * cw2_ref[...], axis=1, keepdims=True)
    trans = jnp.clip(cd * cm, -100.0, 100.0)
    lane = lax.broadcasted_iota(jnp.int32, (1, _CP), 1)
    ef_ref[...] = ef
    tr_ref[...] = jnp.where(lane == 3, 1.0, trans)


def _edge_call(pa, pb, cd, attr, wr, wattr, w2, b2, cw1, cb1, cw2r):
    g = _E // _BE
    return pl.pallas_call(
        _edge_body,
        grid=(g,),
        in_specs=[
            pl.BlockSpec((_BE, _H), lambda i: (i, 0)),
            pl.BlockSpec((_BE, _H), lambda i: (i, 0)),
            pl.BlockSpec((_BE, _CP), lambda i: (i, 0)),
            pl.BlockSpec((_BE, 4), lambda i: (i, 0)),
            pl.BlockSpec((1, _H), lambda i: (0, 0)),
            pl.BlockSpec((4, _H), lambda i: (0, 0)),
            pl.BlockSpec((_H, _H), lambda i: (0, 0)),
            pl.BlockSpec((1, _H), lambda i: (0, 0)),
            pl.BlockSpec((_H, _H), lambda i: (0, 0)),
            pl.BlockSpec((1, _H), lambda i: (0, 0)),
            pl.BlockSpec((1, _H), lambda i: (0, 0)),
        ],
        out_specs=[
            pl.BlockSpec((_BE, _H), lambda i: (i, 0)),
            pl.BlockSpec((_BE, _CP), lambda i: (i, 0)),
        ],
        out_shape=[
            jax.ShapeDtypeStruct((_E, _H), _f32),
            jax.ShapeDtypeStruct((_E, _CP), _f32),
        ],
    )(pa, pb, cd, attr, wr, wattr, w2, b2, cw1, cb1, cw2r)


def _post_body(h_ref, coord_ref, vel_ref, ef0_ref, ef1_ref, tr0_ref, tr1_ref,
               nw1a_ref, nw1b_ref, nb1_ref, nw2_ref, nb2_ref,
               vw1_ref, vb1_ref, vw2_ref, vb2_ref,
               h_out, coord_out):
    h = h_ref[...]
    nagg = ef0_ref[...] + ef1_ref[...]
    tr = tr0_ref[...] + tr1_ref[...]
    cnt = jnp.clip(tr[:, 3:4], 1.0, None)
    agg = tr / cnt
    lane = lax.broadcasted_iota(jnp.int32, (1, _CP), 1)
    agg = jnp.where(lane == 3, 0.0, agg)
    vm = (jnp.sum(_silu(jnp.dot(h, vw1_ref[...], preferred_element_type=_f32)
                        + vb1_ref[...]) * vw2_ref[...], axis=1, keepdims=True)
          + vb2_ref[...])
    coord_out[...] = coord_ref[...] + agg + vm * vel_ref[...]
    t = _silu(jnp.dot(h, nw1a_ref[...], preferred_element_type=_f32)
              + jnp.dot(nagg, nw1b_ref[...], preferred_element_type=_f32)
              + nb1_ref[...])
    h_out[...] = jnp.dot(t, nw2_ref[...], preferred_element_type=_f32) + nb2_ref[...]


def _post_call(h1, coord, vel, ef0, ef1, tr0, tr1,
               nw1a, nw1b, nb1, nw2, nb2, vw1, vb1, vw2r, vb2):
    g = h1.shape[0] // _BN
    nspec = pl.BlockSpec((_BN, _H), lambda i: (i, 0))
    cspec = pl.BlockSpec((_BN, _CP), lambda i: (i, 0))
    wspec = pl.BlockSpec((_H, _H), lambda i: (0, 0))
    bspec = pl.BlockSpec((1, _H), lambda i: (0, 0))
    return pl.pallas_call(
        _post_body,
        grid=(g,),
        in_specs=[nspec, cspec, cspec, nspec, nspec, cspec, cspec,
                  wspec, wspec, bspec, wspec, bspec,
                  wspec, bspec, bspec, pl.BlockSpec((1, 1), lambda i: (0, 0))],
        out_specs=[nspec, cspec],
        out_shape=[
            jax.ShapeDtypeStruct((h1.shape[0], _H), _f32),
            jax.ShapeDtypeStruct((h1.shape[0], _CP), _f32),
        ],
    )(h1, coord, vel, ef0, ef1, tr0, tr1,
      nw1a, nw1b, nb1, nw2, nb2, vw1, vb1, vw2r, vb2)


_MESH = plsc.VectorSubcoreMesh(core_axis_name="c", subcore_axis_name="s")


@functools.partial(
    pl.kernel,
    mesh=_MESH,
    out_type=[
        jax.ShapeDtypeStruct((_E, _H), _f32),
        jax.ShapeDtypeStruct((_E, _H), _f32),
        jax.ShapeDtypeStruct((_E, _CP), _f32),
    ],
    scratch_types=[
        pltpu.VMEM((_EC,), jnp.int32),
        pltpu.VMEM((_EC,), jnp.int32),
        pltpu.VMEM((_EC, _H), _f32),
        pltpu.VMEM((_EC, _H), _f32),
        pltpu.VMEM((_EC, _H), _f32),
        pltpu.VMEM((_EC, _H), _f32),
        pltpu.VMEM((_EC, _CP), _f32),
        pltpu.SemaphoreType.DMA,
    ],
)
def _sc_gather(a_hbm, b_hbm, c_hbm, row_hbm, col_hbm,
               pa_hbm, pb_hbm, cd_hbm,
               idx_r, idx_c, buf_a, buf_b, buf_r, buf_c, buf_d, sem):
    cid = lax.axis_index("c")
    sid = lax.axis_index("s")
    wid = sid * _NC + cid

    @pl.loop(0, _NCH)
    def _(i):
        base = wid * _EPW + i * _EC
        sl = pl.ds(base, _EC)
        pltpu.sync_copy(row_hbm.at[sl], idx_r)
        pltpu.sync_copy(col_hbm.at[sl], idx_c)
        ca = pltpu.async_copy(a_hbm.at[idx_r], buf_a, sem)
        cb = pltpu.async_copy(b_hbm.at[idx_c], buf_b, sem)
        cr = pltpu.async_copy(c_hbm.at[idx_r], buf_r, sem)
        cc = pltpu.async_copy(c_hbm.at[idx_c], buf_c, sem)
        ca.wait()
        cb.wait()
        cr.wait()
        cc.wait()

        @pl.loop(0, _EC)
        def _(e):
            buf_d[e, pl.ds(0, _CP)] = (buf_r[e, pl.ds(0, _CP)]
                                       - buf_c[e, pl.ds(0, _CP)])

        pltpu.sync_copy(buf_a, pa_hbm.at[sl])
        pltpu.sync_copy(buf_b, pb_hbm.at[sl])
        pltpu.sync_copy(buf_d, cd_hbm.at[sl])


_ST = 624
_TAIL = _N - _ST * _NS
_SCK = 104


def _make_scatter(width):
    @functools.partial(
        pl.kernel,
        mesh=_MESH,
        out_type=jax.ShapeDtypeStruct((_NC, _N, width), _f32),
        scratch_types=[
            pltpu.VMEM((_EC,), jnp.int32),
            pltpu.VMEM((_EC, width), _f32),
            pltpu.VMEM((_SCK, width), _f32),
            pltpu.VMEM_SHARED((_N, width), _f32),
            pltpu.SemaphoreType.DMA,
        ],
    )
    def _scatter(v_hbm, row_hbm, z_hbm, out_hbm, idx, buf, stg, acc, sem):
        cid = lax.axis_index("c")
        sid = lax.axis_index("s")

        pltpu.sync_copy(z_hbm.at[pl.ds(0, _SCK)], stg)

        @pl.loop(0, _ST // _SCK)
        def _(k):
            pltpu.sync_copy(stg, acc.at[pl.ds(sid * _ST + k * _SCK, _SCK)])

        @pl.when(sid == 0)
        def _():
            pltpu.sync_copy(stg.at[pl.ds(0, _TAIL)],
                            acc.at[pl.ds(_ST * _NS, _TAIL)])

        plsc.subcore_barrier()

        @pl.loop(0, _NCH)
        def _(i):
            base = cid * (_E // _NC) + sid * _EPW + i * _EC
            sl = pl.ds(base, _EC)
            pltpu.sync_copy(row_hbm.at[sl], idx)
            pltpu.sync_copy(v_hbm.at[sl], buf)
            pltpu.sync_copy(buf, acc.at[idx], add=True)

        plsc.subcore_barrier()

        @pl.loop(0, _ST // _SCK)
        def _(k):
            ds = pl.ds(sid * _ST + k * _SCK, _SCK)
            pltpu.sync_copy(acc.at[ds], stg)
            pltpu.sync_copy(stg, out_hbm.at[cid].at[ds])

        @pl.when(sid == 0)
        def _():
            tail = pl.ds(_ST * _NS, _TAIL)
            pltpu.sync_copy(acc.at[tail], stg.at[pl.ds(0, _TAIL)])
            pltpu.sync_copy(stg.at[pl.ds(0, _TAIL)], out_hbm.at[cid].at[tail])

    return _scatter


_sc_scatter_ef = _make_scatter(_H)
_sc_scatter_tr = _make_scatter(_CP)


def kernel(h, x, edges, vel, edge_attr, emb_w, emb_b, edge_w1, edge_b1,
           edge_w2, edge_b2, node_w1, node_b1, node_w2, node_b2,
           coord_w1, coord_b1, coord_w2, vel_w1, vel_b1, vel_w2, vel_b2):
    row = edges[0]
    col = edges[1]

    pad = jnp.zeros((_N, _CP - 3), _f32)
    coord = jnp.concatenate([x, pad], axis=1)
    velp = jnp.concatenate([vel, pad], axis=1)
    cpad = jnp.zeros((_N, _H - _CP), _f32)
    zf = jnp.zeros((_N, _H), _f32)
    zt = jnp.zeros((_N, _CP), _f32)

    h1 = _emb_call(h, emb_w, emb_b.reshape(1, _H))

    for i in range(_L):
        w1 = edge_w1[i]
        w_row = w1[:_H]
        w_col = w1[_H:2 * _H]
        wr = w1[2 * _H:2 * _H + 1]
        wattr = w1[2 * _H + 1:]
        a, b = _pre_call(h1, w_row, w_col, edge_b1[i].reshape(1, _H))
        c128 = jnp.concatenate([coord, cpad], axis=1)
        pa, pb, cd = _sc_gather(a, b, c128, row, col)
        ef, tr = _edge_call(pa, pb, cd, edge_attr,
                            wr, wattr, edge_w2[i], edge_b2[i].reshape(1, _H),
                            coord_w1[i], coord_b1[i].reshape(1, _H),
                            coord_w2[i].reshape(1, _H))
        oef = _sc_scatter_ef(ef, row, zf)
        otr = _sc_scatter_tr(tr, row, zt)
        h1, coord = _post_call(
            h1, coord, velp, oef[0], oef[1], otr[0], otr[1],
            node_w1[i][:_H], node_w1[i][_H:], node_b1[i].reshape(1, _H),
            node_w2[i], node_b2[i].reshape(1, _H),
            vel_w1[i], vel_b1[i].reshape(1, _H),
            vel_w2[i].reshape(1, _H), vel_b2[i].reshape(1, 1))

    return coord[:, :3]
